# Initial kernel scaffold; baseline (speedup 1.0000x reference)
import jax
import jax.numpy as jnp
from jax.experimental import pallas as pl


def kernel(pred_distri, pred_scores, anchor_points, gt_labels, gt_bboxes, mask_gt):
    raise NotImplementedError("write your pallas kernel here")


# speedup vs baseline: 3.7032x; 3.7032x over previous
import math

import jax
import jax.numpy as jnp
from jax.experimental import pallas as pl
from jax.experimental.pallas import tpu as pltpu

_B, _A, _NC, _REG_MAX, _NMAX, _TOPK = 16, 8400, 80, 16, 32, 10
_ALPHA, _BETA, _EPS = 0.5, 6.0, 1e-9
_CEPS = 1e-7

_TA = 840
_NT = _A // _TA

_ATAN_COEFFS = (
    9.9999999773e-01, -3.3333285376e-01, 1.9998315719e-01, -1.4262475385e-01,
    1.0944970499e-01, -8.3862066348e-02, 5.7709186551e-02, -3.0965612942e-02,
    1.0815613194e-02, -1.7742115459e-03,
)


def _atan(x):
    a = jnp.abs(x)
    inv = a > 1.0
    r = jnp.where(inv, 1.0 / jnp.maximum(a, 1e-30), a)
    z = r * r
    p = jnp.full_like(z, _ATAN_COEFFS[-1])
    for c in _ATAN_COEFFS[-2::-1]:
        p = p * z + c
    p = p * r
    res = jnp.where(inv, (math.pi / 2) - p, p)
    return jnp.where(x < 0, -res, res)


def _loss_kernel(pd_ref, ps_ref, anc_ref, gtb_ref, lab_ref, mgt_ref, out_ref,
                 al_s, ov_s, mp_s, aux_s, cm_s):
    b = pl.program_id(0)
    f32 = jnp.float32

    gx1 = gtb_ref[0, 0:1, :]
    gy1 = gtb_ref[0, 1:2, :]
    gx2 = gtb_ref[0, 2:3, :]
    gy2 = gtb_ref[0, 3:4, :]
    lab = lab_ref[0]
    mgt = mgt_ref[0]

    w2 = gx2 - gx1
    h2 = gy2 - gy1 + _CEPS
    at2 = _atan(w2 / (h2 + _CEPS))

    iota16 = jax.lax.broadcasted_iota(jnp.int32, (_TA, _REG_MAX), 1).astype(f32)
    iota32 = jax.lax.broadcasted_iota(jnp.int32, (_TA, _NMAX), 1).astype(f32)
    iota80 = jax.lax.broadcasted_iota(jnp.int32, (_TA, _NC), 1).astype(f32)
    iota_nc = jax.lax.broadcasted_iota(jnp.int32, (_NC, _NMAX), 0).astype(f32)
    onehot_lab = (iota_nc == lab).astype(f32)
    lane = jax.lax.broadcasted_iota(jnp.int32, (1, 128), 1)

    @pl.when(b == 0)
    def _():
        out_ref[...] = jnp.zeros((1, 128), f32)

    cm_s[...] = jnp.zeros((8, 128), f32)

    def pass1(t, _):
        sl = pl.ds(t * _TA, _TA)
        pd = pd_ref[0, sl, :]
        ps = ps_ref[0, sl, :]
        ax = anc_ref[sl, 0:1]
        ay = anc_ref[sl, 1:2]

        ltrb, logz = [], []
        for c in range(4):
            x = pd[:, c * _REG_MAX:(c + 1) * _REG_MAX]
            m = jnp.max(x, axis=1, keepdims=True)
            e = jnp.exp(x - m)
            s = jnp.sum(e, axis=1, keepdims=True)
            ltrb.append(jnp.sum(e * iota16, axis=1, keepdims=True) / s)
            logz.append(m + jnp.log(s))

        px1 = ax - ltrb[0]
        py1 = ay - ltrb[1]
        px2 = ax + ltrb[2]
        py2 = ay + ltrb[3]

        sig = 1.0 / (1.0 + jnp.exp(-ps))
        bsc = jnp.dot(sig, onehot_lab, preferred_element_type=f32)

        sp_sum = jnp.sum(jnp.maximum(ps, 0.0)
                         + jnp.log1p(jnp.exp(-jnp.abs(ps))))

        dmin = jnp.minimum(jnp.minimum(ax - gx1, ay - gy1),
                           jnp.minimum(gx2 - ax, gy2 - ay))
        mask = (dmin > _EPS).astype(f32) * mgt

        w1 = px2 - px1
        h1 = py2 - py1 + _CEPS
        at1 = _atan(w1 / (h1 + _CEPS))
        inter = (jnp.maximum(jnp.minimum(px2, gx2) - jnp.maximum(px1, gx1),
                             0.0)
                 * jnp.maximum(jnp.minimum(py2, gy2) - jnp.maximum(py1, gy1),
                               0.0))
        union = w1 * h1 + w2 * h2 - inter + _CEPS
        iou = inter / union
        cw = jnp.maximum(px2, gx2) - jnp.minimum(px1, gx1)
        ch = jnp.maximum(py2, gy2) - jnp.minimum(py1, gy1)
        c2 = cw * cw + ch * ch + _CEPS
        rho2 = ((gx1 + gx2 - px1 - px2) ** 2
                + (gy1 + gy2 - py1 - py2) ** 2) * 0.25
        dd = at2 - at1
        v = (4.0 / math.pi ** 2) * dd * dd
        alpha = v / (v - iou + (1.0 + _CEPS))
        ciou = iou - (rho2 / c2 + v * alpha)
        ov = jnp.clip(ciou, -1.0, 1.0) * mask

        ovr = jnp.maximum(ov, 0.0)
        ovr2 = ovr * ovr
        align = jnp.sqrt(bsc * mask) * (ovr2 * ovr2 * ovr2)

        al_s[sl, :] = align
        ov_s[sl, :] = ov
        aux_s[sl, 0:1] = px1
        aux_s[sl, 1:2] = py1
        aux_s[sl, 2:3] = px2
        aux_s[sl, 3:4] = py2
        aux_s[sl, 4:5] = at1
        aux_s[sl, 5:6] = logz[0]
        aux_s[sl, 6:7] = logz[1]
        aux_s[sl, 7:8] = logz[2]
        aux_s[sl, 8:9] = logz[3]

        out_ref[...] += jnp.where(lane == 1, sp_sum, 0.0)
        return 0

    jax.lax.fori_loop(0, _NT, pass1, 0)

    work = al_s[...]
    for _ in range(_TOPK - 1):
        cur = jnp.max(work, axis=0, keepdims=True)
        work = jnp.where(work >= cur, -1.0, work)
    thr = jnp.max(work, axis=0, keepdims=True)

    def pass2(t, _):
        sl = pl.ds(t * _TA, _TA)
        align = al_s[sl, :]
        ov = ov_s[sl, :]

        pos0 = jnp.logical_and(align >= thr, align > _EPS).astype(f32)
        fg0 = jnp.sum(pos0, axis=1, keepdims=True)
        multi = fg0 > 1.0

        mxo = jnp.max(ov, axis=1, keepdims=True)
        fidx = jnp.min(jnp.where(ov >= mxo, iota32, float(_NMAX)), axis=1,
                       keepdims=True)
        ismax = (iota32 == fidx).astype(f32)
        mask_pos = jnp.where(multi, ismax, pos0)
        mp_s[sl, :] = mask_pos

        am_col = jnp.max(align * mask_pos, axis=0, keepdims=True)
        ov_col = jnp.max(ov * mask_pos, axis=0, keepdims=True)
        cm_s[0:1, 0:_NMAX] = jnp.maximum(cm_s[0:1, 0:_NMAX], am_col)
        cm_s[1:2, 0:_NMAX] = jnp.maximum(cm_s[1:2, 0:_NMAX], ov_col)
        return 0

    jax.lax.fori_loop(0, _NT, pass2, 0)

    pos_align = cm_s[0:1, 0:_NMAX]
    pos_ov = cm_s[1:2, 0:_NMAX]
    ratio = pos_ov / (pos_align + _EPS)

    def pass3(t, _):
        sl = pl.ds(t * _TA, _TA)
        align = al_s[sl, :]
        mask_pos = mp_s[sl, :]
        ps = ps_ref[0, sl, :]
        pd = pd_ref[0, sl, :]
        ax = anc_ref[sl, 0:1]
        ay = anc_ref[sl, 1:2]
        px1 = aux_s[sl, 0:1]
        py1 = aux_s[sl, 1:2]
        px2 = aux_s[sl, 2:3]
        py2 = aux_s[sl, 3:4]
        at1 = aux_s[sl, 4:5]

        mxp = jnp.max(mask_pos, axis=1, keepdims=True)
        sidx = jnp.min(jnp.where(mask_pos >= mxp, iota32, float(_NMAX)),
                       axis=1, keepdims=True)
        sel = (iota32 == sidx).astype(f32)
        fg = (mxp > 0.0).astype(f32)

        def _sel(row):
            return jnp.sum(sel * row, axis=1, keepdims=True)

        tx1, ty1, tx2, ty2 = _sel(gx1), _sel(gy1), _sel(gx2), _sel(gy2)
        tlab = _sel(lab)
        tat2 = _sel(at2)

        ampos = align * mask_pos
        norm = jnp.max(ampos * ratio, axis=1, keepdims=True)

        fgn = fg * norm
        weight = jnp.clip(fgn, 1e-6, None)
        ts_sum = jnp.sum(fgn)

        psel = jnp.sum(jnp.where(iota80 == tlab, ps, 0.0), axis=1,
                       keepdims=True)
        neg_bce = jnp.sum(psel * fgn)

        w1 = px2 - px1
        h1 = py2 - py1 + _CEPS
        tw = tx2 - tx1
        th = ty2 - ty1 + _CEPS
        inter = (jnp.maximum(jnp.minimum(px2, tx2) - jnp.maximum(px1, tx1),
                             0.0)
                 * jnp.maximum(jnp.minimum(py2, ty2) - jnp.maximum(py1, ty1),
                               0.0))
        union = w1 * h1 + tw * th - inter + _CEPS
        iou = inter / union
        cw = jnp.maximum(px2, tx2) - jnp.minimum(px1, tx1)
        ch = jnp.maximum(py2, ty2) - jnp.minimum(py1, ty1)
        c2 = cw * cw + ch * ch + _CEPS
        rho2 = ((tx1 + tx2 - px1 - px2) ** 2
                + (ty1 + ty2 - py1 - py2) ** 2) * 0.25
        dd = tat2 - at1
        v = (4.0 / math.pi ** 2) * dd * dd
        alpha = v / (v - iou + (1.0 + _CEPS))
        ciou = iou - (rho2 / c2 + v * alpha)
        iou_t = jnp.clip(ciou, -1.0, 1.0)
        iou_sum = jnp.sum((1.0 - iou_t) * weight * fg)

        tltrb = [jnp.clip(tv, 0.0, _REG_MAX - 1.01)
                 for tv in (ax - tx1, ay - ty1, tx2 - ax, ty2 - ay)]
        acc = jnp.zeros((_TA, 1), f32)
        for c in range(4):
            tv = tltrb[c]
            tlf = jnp.clip(jnp.floor(tv), 0.0, float(_REG_MAX - 2))
            trf = tlf + 1.0
            wl = jnp.clip(trf - tv, 0.0, 1.0)
            wr = jnp.clip(tv - tlf, 0.0, 1.0)
            lp = (pd[:, c * _REG_MAX:(c + 1) * _REG_MAX]
                  - aux_s[sl, 5 + c:6 + c])
            left = -jnp.sum(jnp.where(iota16 == tlf, lp, 0.0), axis=1,
                            keepdims=True)
            right = -jnp.sum(jnp.where(iota16 == trf, lp, 0.0), axis=1,
                             keepdims=True)
            acc = acc + left * wl + right * wr
        dfl = jnp.clip(acc * 0.25, None, 100.0)
        dfl_sum = jnp.sum(dfl * weight * fg)

        out_ref[...] += (jnp.where(lane == 0, ts_sum, 0.0)
                         + jnp.where(lane == 1, -neg_bce, 0.0)
                         + jnp.where(lane == 2, iou_sum, 0.0)
                         + jnp.where(lane == 3, dfl_sum, 0.0))
        return 0

    jax.lax.fori_loop(0, _NT, pass3, 0)


def kernel(pred_distri, pred_scores, anchor_points, gt_labels, gt_bboxes,
           mask_gt):
    gtb_t = jnp.transpose(gt_bboxes, (0, 2, 1))
    lab_f = gt_labels[..., 0].astype(jnp.float32)[:, None, :]
    mgt_t = jnp.transpose(mask_gt, (0, 2, 1))

    out = pl.pallas_call(
        _loss_kernel,
        grid=(_B,),
        in_specs=[
            pl.BlockSpec((1, _A, 4 * _REG_MAX), lambda b: (b, 0, 0)),
            pl.BlockSpec((1, _A, _NC), lambda b: (b, 0, 0)),
            pl.BlockSpec((_A, 2), lambda b: (0, 0)),
            pl.BlockSpec((1, 4, _NMAX), lambda b: (b, 0, 0)),
            pl.BlockSpec((1, 1, _NMAX), lambda b: (b, 0, 0)),
            pl.BlockSpec((1, 1, _NMAX), lambda b: (b, 0, 0)),
        ],
        out_specs=pl.BlockSpec((1, 128), lambda b: (0, 0)),
        out_shape=jax.ShapeDtypeStruct((1, 128), jnp.float32),
        scratch_shapes=[
            pltpu.VMEM((_A, _NMAX), jnp.float32),
            pltpu.VMEM((_A, _NMAX), jnp.float32),
            pltpu.VMEM((_A, _NMAX), jnp.float32),
            pltpu.VMEM((_A, 16), jnp.float32),
            pltpu.VMEM((8, 128), jnp.float32),
        ],
    )(pred_distri, pred_scores, anchor_points, gtb_t, lab_f, mgt_t)

    s = out[0]
    tss = jnp.maximum(s[0], 1.0)
    loss_iou = jnp.clip(s[2] / tss, None, 100.0)
    loss_cls = s[1] / tss
    loss_dfl = jnp.clip(s[3] / tss, None, 100.0)
    return jnp.stack([loss_iou, loss_cls, loss_dfl])


# speedup vs baseline: 20.6585x; 5.5785x over previous
import math

import jax
import jax.numpy as jnp
from jax.experimental import pallas as pl
from jax.experimental.pallas import tpu as pltpu

_B, _A, _NC, _REG_MAX, _NMAX, _TOPK = 16, 8400, 80, 16, 32, 10
_ALPHA, _BETA, _EPS = 0.5, 6.0, 1e-9
_CEPS = 1e-7

_NT = 6
_TA = _A // _NT

_ATAN_COEFFS = (
    9.9999999773e-01, -3.3333285376e-01, 1.9998315719e-01, -1.4262475385e-01,
    1.0944970499e-01, -8.3862066348e-02, 5.7709186551e-02, -3.0965612942e-02,
    1.0815613194e-02, -1.7742115459e-03,
)


def _atan(x):
    a = jnp.abs(x)
    inv = a > 1.0
    r = jnp.where(inv, 1.0 / jnp.maximum(a, 1e-30), a)
    z = r * r
    p = jnp.full_like(z, _ATAN_COEFFS[-1])
    for c in _ATAN_COEFFS[-2::-1]:
        p = p * z + c
    p = p * r
    res = jnp.where(inv, (math.pi / 2) - p, p)
    return jnp.where(x < 0, -res, res)


def _loss_kernel(pd_ref, ps_ref, anc_ref, gtb_ref, lab_ref, mgt_ref, out_ref,
                 al_s, ov_s, mp_s, aux_s, cm_s):
    b = pl.program_id(0)
    f32 = jnp.float32

    gx1 = gtb_ref[0, :, 0:1]
    gy1 = gtb_ref[0, :, 1:2]
    gx2 = gtb_ref[0, :, 2:3]
    gy2 = gtb_ref[0, :, 3:4]
    lab = lab_ref[0]
    mgt = mgt_ref[0]

    w2 = gx2 - gx1
    h2 = gy2 - gy1 + _CEPS
    at2 = _atan(w2 / (h2 + _CEPS))

    i16s = jax.lax.broadcasted_iota(jnp.int32, (_REG_MAX, _TA), 0).astype(f32)
    i32s = jax.lax.broadcasted_iota(jnp.int32, (_NMAX, _TA), 0).astype(f32)
    i80s = jax.lax.broadcasted_iota(jnp.int32, (_NC, _TA), 0).astype(f32)
    iota_nc = jax.lax.broadcasted_iota(jnp.int32, (_NMAX, _NC), 1).astype(f32)
    onehot_lab = (iota_nc == lab).astype(f32)
    lane = jax.lax.broadcasted_iota(jnp.int32, (1, 128), 1)

    @pl.when(b == 0)
    def _():
        out_ref[...] = jnp.zeros((1, 128), f32)

    cm_s[...] = jnp.zeros((_NMAX, 128), f32)

    def pass1(t, _):
        pd = pd_ref[0, t]
        ps = ps_ref[0, t]
        ax = anc_ref[t, 0:1, :]
        ay = anc_ref[t, 1:2, :]

        ltrb, logz = [], []
        for c in range(4):
            x = pd[c * _REG_MAX:(c + 1) * _REG_MAX, :]
            m = jnp.max(x, axis=0, keepdims=True)
            e = jnp.exp(x - m)
            s = jnp.sum(e, axis=0, keepdims=True)
            ltrb.append(jnp.sum(e * i16s, axis=0, keepdims=True) / s)
            logz.append(m + jnp.log(s))

        px1 = ax - ltrb[0]
        py1 = ay - ltrb[1]
        px2 = ax + ltrb[2]
        py2 = ay + ltrb[3]

        sig = 1.0 / (1.0 + jnp.exp(-ps))
        bsc = jnp.dot(onehot_lab, sig, preferred_element_type=f32)

        sp_sum = jnp.sum(jnp.maximum(ps, 0.0)
                         + jnp.log1p(jnp.exp(-jnp.abs(ps))))

        dmin = jnp.minimum(jnp.minimum(ax - gx1, ay - gy1),
                           jnp.minimum(gx2 - ax, gy2 - ay))
        mask = (dmin > _EPS).astype(f32) * mgt

        w1 = px2 - px1
        h1 = py2 - py1 + _CEPS
        at1 = _atan(w1 / (h1 + _CEPS))
        inter = (jnp.maximum(jnp.minimum(px2, gx2) - jnp.maximum(px1, gx1),
                             0.0)
                 * jnp.maximum(jnp.minimum(py2, gy2) - jnp.maximum(py1, gy1),
                               0.0))
        union = w1 * h1 + w2 * h2 - inter + _CEPS
        iou = inter / union
        cw = jnp.maximum(px2, gx2) - jnp.minimum(px1, gx1)
        ch = jnp.maximum(py2, gy2) - jnp.minimum(py1, gy1)
        c2 = cw * cw + ch * ch + _CEPS
        rho2 = ((gx1 + gx2 - px1 - px2) ** 2
                + (gy1 + gy2 - py1 - py2) ** 2) * 0.25
        dd = at2 - at1
        v = (4.0 / math.pi ** 2) * dd * dd
        alpha = v / (v - iou + (1.0 + _CEPS))
        ciou = iou - (rho2 / c2 + v * alpha)
        ov = jnp.clip(ciou, -1.0, 1.0) * mask

        ovr = jnp.maximum(ov, 0.0)
        ovr2 = ovr * ovr
        align = jnp.sqrt(bsc * mask) * (ovr2 * ovr2 * ovr2)

        al_s[t] = align
        ov_s[t] = ov
        aux_s[t, 0:1, :] = px1
        aux_s[t, 1:2, :] = py1
        aux_s[t, 2:3, :] = px2
        aux_s[t, 3:4, :] = py2
        aux_s[t, 4:5, :] = at1
        aux_s[t, 5:6, :] = logz[0]
        aux_s[t, 6:7, :] = logz[1]
        aux_s[t, 7:8, :] = logz[2]
        aux_s[t, 8:9, :] = logz[3]

        out_ref[...] += jnp.where(lane == 1, sp_sum, 0.0)
        return 0

    jax.lax.fori_loop(0, _NT, pass1, 0)

    work = al_s[...]
    for _ in range(_TOPK - 1):
        cur = jnp.max(work, axis=(0, 2))[None, :, None]
        work = jnp.where(work >= cur, -1.0, work)
    thr = jnp.max(work, axis=(0, 2))[:, None]

    def pass2(t, _):
        align = al_s[t]
        ov = ov_s[t]

        pos0 = jnp.logical_and(align >= thr, align > _EPS).astype(f32)
        fg0 = jnp.sum(pos0, axis=0, keepdims=True)
        multi = fg0 > 1.0

        mxo = jnp.max(ov, axis=0, keepdims=True)
        fidx = jnp.min(jnp.where(ov >= mxo, i32s, float(_NMAX)), axis=0,
                       keepdims=True)
        ismax = (i32s == fidx).astype(f32)
        mask_pos = jnp.where(multi, ismax, pos0)
        mp_s[t] = mask_pos

        am_col = jnp.max(align * mask_pos, axis=1, keepdims=True)
        ov_col = jnp.max(ov * mask_pos, axis=1, keepdims=True)
        cm_s[:, 0:1] = jnp.maximum(cm_s[:, 0:1], am_col)
        cm_s[:, 1:2] = jnp.maximum(cm_s[:, 1:2], ov_col)
        return 0

    jax.lax.fori_loop(0, _NT, pass2, 0)

    pos_align = cm_s[:, 0:1]
    pos_ov = cm_s[:, 1:2]
    ratio = pos_ov / (pos_align + _EPS)

    def pass3(t, _):
        align = al_s[t]
        mask_pos = mp_s[t]
        ps = ps_ref[0, t]
        pd = pd_ref[0, t]
        ax = anc_ref[t, 0:1, :]
        ay = anc_ref[t, 1:2, :]
        px1 = aux_s[t, 0:1, :]
        py1 = aux_s[t, 1:2, :]
        px2 = aux_s[t, 2:3, :]
        py2 = aux_s[t, 3:4, :]
        at1 = aux_s[t, 4:5, :]

        mxp = jnp.max(mask_pos, axis=0, keepdims=True)
        sidx = jnp.min(jnp.where(mask_pos >= mxp, i32s, float(_NMAX)),
                       axis=0, keepdims=True)
        sel = (i32s == sidx).astype(f32)
        fg = (mxp > 0.0).astype(f32)

        def _sel(col):
            return jnp.sum(sel * col, axis=0, keepdims=True)

        tx1, ty1, tx2, ty2 = _sel(gx1), _sel(gy1), _sel(gx2), _sel(gy2)
        tlab = _sel(lab)
        tat2 = _sel(at2)

        ampos = align * mask_pos
        norm = jnp.max(ampos * ratio, axis=0, keepdims=True)

        fgn = fg * norm
        weight = jnp.clip(fgn, 1e-6, None)
        ts_sum = jnp.sum(fgn)

        psel = jnp.sum(jnp.where(i80s == tlab, ps, 0.0), axis=0,
                       keepdims=True)
        neg_bce = jnp.sum(psel * fgn)

        w1 = px2 - px1
        h1 = py2 - py1 + _CEPS
        tw = tx2 - tx1
        th = ty2 - ty1 + _CEPS
        inter = (jnp.maximum(jnp.minimum(px2, tx2) - jnp.maximum(px1, tx1),
                             0.0)
                 * jnp.maximum(jnp.minimum(py2, ty2) - jnp.maximum(py1, ty1),
                               0.0))
        union = w1 * h1 + tw * th - inter + _CEPS
        iou = inter / union
        cw = jnp.maximum(px2, tx2) - jnp.minimum(px1, tx1)
        ch = jnp.maximum(py2, ty2) - jnp.minimum(py1, ty1)
        c2 = cw * cw + ch * ch + _CEPS
        rho2 = ((tx1 + tx2 - px1 - px2) ** 2
                + (ty1 + ty2 - py1 - py2) ** 2) * 0.25
        dd = tat2 - at1
        v = (4.0 / math.pi ** 2) * dd * dd
        alpha = v / (v - iou + (1.0 + _CEPS))
        ciou = iou - (rho2 / c2 + v * alpha)
        iou_t = jnp.clip(ciou, -1.0, 1.0)
        iou_sum = jnp.sum((1.0 - iou_t) * weight * fg)

        tltrb = [jnp.clip(tv, 0.0, _REG_MAX - 1.01)
                 for tv in (ax - tx1, ay - ty1, tx2 - ax, ty2 - ay)]
        acc = jnp.zeros((1, _TA), f32)
        for c in range(4):
            tv = tltrb[c]
            tlf = jnp.clip(jnp.floor(tv), 0.0, float(_REG_MAX - 2))
            trf = tlf + 1.0
            wl = jnp.clip(trf - tv, 0.0, 1.0)
            wr = jnp.clip(tv - tlf, 0.0, 1.0)
            lp = (pd[c * _REG_MAX:(c + 1) * _REG_MAX, :]
                  - aux_s[t, 5 + c:6 + c, :])
            left = -jnp.sum(jnp.where(i16s == tlf, lp, 0.0), axis=0,
                            keepdims=True)
            right = -jnp.sum(jnp.where(i16s == trf, lp, 0.0), axis=0,
                             keepdims=True)
            acc = acc + left * wl + right * wr
        dfl = jnp.clip(acc * 0.25, None, 100.0)
        dfl_sum = jnp.sum(dfl * weight * fg)

        out_ref[...] += (jnp.where(lane == 0, ts_sum, 0.0)
                         + jnp.where(lane == 1, -neg_bce, 0.0)
                         + jnp.where(lane == 2, iou_sum, 0.0)
                         + jnp.where(lane == 3, dfl_sum, 0.0))
        return 0

    jax.lax.fori_loop(0, _NT, pass3, 0)


def kernel(pred_distri, pred_scores, anchor_points, gt_labels, gt_bboxes,
           mask_gt):
    f32 = jnp.float32
    pd_t = pred_distri.reshape(_B, _NT, _TA, 4 * _REG_MAX).transpose(0, 1, 3, 2)
    ps_t = pred_scores.reshape(_B, _NT, _TA, _NC).transpose(0, 1, 3, 2)
    anc_t = anchor_points.reshape(_NT, _TA, 2).transpose(0, 2, 1)
    lab_f = gt_labels.astype(f32)

    out = pl.pallas_call(
        _loss_kernel,
        grid=(_B,),
        in_specs=[
            pl.BlockSpec((1, _NT, 4 * _REG_MAX, _TA), lambda b: (b, 0, 0, 0)),
            pl.BlockSpec((1, _NT, _NC, _TA), lambda b: (b, 0, 0, 0)),
            pl.BlockSpec((_NT, 2, _TA), lambda b: (0, 0, 0)),
            pl.BlockSpec((1, _NMAX, 4), lambda b: (b, 0, 0)),
            pl.BlockSpec((1, _NMAX, 1), lambda b: (b, 0, 0)),
            pl.BlockSpec((1, _NMAX, 1), lambda b: (b, 0, 0)),
        ],
        out_specs=pl.BlockSpec((1, 128), lambda b: (0, 0)),
        out_shape=jax.ShapeDtypeStruct((1, 128), f32),
        scratch_shapes=[
            pltpu.VMEM((_NT, _NMAX, _TA), f32),
            pltpu.VMEM((_NT, _NMAX, _TA), f32),
            pltpu.VMEM((_NT, _NMAX, _TA), f32),
            pltpu.VMEM((_NT, 16, _TA), f32),
            pltpu.VMEM((_NMAX, 128), f32),
        ],
    )(pd_t, ps_t, anc_t, gt_bboxes, lab_f, mask_gt)

    s = out[0]
    tss = jnp.maximum(s[0], 1.0)
    loss_iou = jnp.clip(s[2] / tss, None, 100.0)
    loss_cls = s[1] / tss
    loss_dfl = jnp.clip(s[3] / tss, None, 100.0)
    return jnp.stack([loss_iou, loss_cls, loss_dfl])


# speedup vs baseline: 24.0948x; 1.1663x over previous
import math

import jax
import jax.numpy as jnp
from jax.experimental import pallas as pl
from jax.experimental.pallas import tpu as pltpu

_B, _A, _NC, _REG_MAX, _NMAX, _TOPK = 16, 8400, 80, 16, 32, 10
_ALPHA, _BETA, _EPS = 0.5, 6.0, 1e-9
_CEPS = 1e-7

_NT = 6
_TA = _A // _NT

_ATAN_COEFFS = (
    9.9999999773e-01, -3.3333285376e-01, 1.9998315719e-01, -1.4262475385e-01,
    1.0944970499e-01, -8.3862066348e-02, 5.7709186551e-02, -3.0965612942e-02,
    1.0815613194e-02, -1.7742115459e-03,
)


def _atan(x):
    a = jnp.abs(x)
    inv = a > 1.0
    r = jnp.where(inv, 1.0 / jnp.maximum(a, 1e-30), a)
    z = r * r
    p = jnp.full_like(z, _ATAN_COEFFS[-1])
    for c in _ATAN_COEFFS[-2::-1]:
        p = p * z + c
    p = p * r
    res = jnp.where(inv, (math.pi / 2) - p, p)
    return jnp.where(x < 0, -res, res)


def _loss_kernel(pd_ref, ps_ref, anc_ref, gtb_ref, lab_ref, mgt_ref, out_ref,
                 al_s, ov_s, mp_s, aux_s, pdt_s, cm_s):
    b = pl.program_id(0)
    f32 = jnp.float32

    gx1 = gtb_ref[0, :, 0:1]
    gy1 = gtb_ref[0, :, 1:2]
    gx2 = gtb_ref[0, :, 2:3]
    gy2 = gtb_ref[0, :, 3:4]
    lab = lab_ref[0]
    mgt = mgt_ref[0]

    w2 = gx2 - gx1
    h2 = gy2 - gy1 + _CEPS
    at2 = _atan(w2 / (h2 + _CEPS))

    i16s = jax.lax.broadcasted_iota(jnp.int32, (_REG_MAX, _TA), 0).astype(f32)
    i32s = jax.lax.broadcasted_iota(jnp.int32, (_NMAX, _TA), 0).astype(f32)
    iota_nc = jax.lax.broadcasted_iota(jnp.int32, (_NMAX, _NC), 1).astype(f32)
    onehot_lab = (iota_nc == lab).astype(f32)
    lane = jax.lax.broadcasted_iota(jnp.int32, (1, 128), 1)

    @pl.when(b == 0)
    def _():
        out_ref[...] = jnp.zeros((1, 128), f32)

    cm_s[...] = jnp.zeros((_NMAX, 128), f32)

    def pass1(t, _):
        sl = pl.ds(t * _TA, _TA)
        pd = jnp.transpose(pd_ref[0, sl, :])
        ps = jnp.transpose(ps_ref[0, sl, :])
        pdt_s[t] = pd
        ax = anc_ref[t, 0:1, :]
        ay = anc_ref[t, 1:2, :]

        ltrb, logz = [], []
        for c in range(4):
            x = pd[c * _REG_MAX:(c + 1) * _REG_MAX, :]
            m = jnp.max(x, axis=0, keepdims=True)
            e = jnp.exp(x - m)
            s = jnp.sum(e, axis=0, keepdims=True)
            ltrb.append(jnp.sum(e * i16s, axis=0, keepdims=True) / s)
            logz.append(m + jnp.log(s))

        px1 = ax - ltrb[0]
        py1 = ay - ltrb[1]
        px2 = ax + ltrb[2]
        py2 = ay + ltrb[3]

        sig = 1.0 / (1.0 + jnp.exp(-ps))
        bsc = jnp.dot(onehot_lab, sig, preferred_element_type=f32)

        sp_sum = jnp.sum(jnp.maximum(ps, 0.0)
                         + jnp.log1p(jnp.exp(-jnp.abs(ps))))

        dmin = jnp.minimum(jnp.minimum(ax - gx1, ay - gy1),
                           jnp.minimum(gx2 - ax, gy2 - ay))
        mask = (dmin > _EPS).astype(f32) * mgt

        w1 = px2 - px1
        h1 = py2 - py1 + _CEPS
        at1 = _atan(w1 / (h1 + _CEPS))
        inter = (jnp.maximum(jnp.minimum(px2, gx2) - jnp.maximum(px1, gx1),
                             0.0)
                 * jnp.maximum(jnp.minimum(py2, gy2) - jnp.maximum(py1, gy1),
                               0.0))
        union = w1 * h1 + w2 * h2 - inter + _CEPS
        iou = inter / union
        cw = jnp.maximum(px2, gx2) - jnp.minimum(px1, gx1)
        ch = jnp.maximum(py2, gy2) - jnp.minimum(py1, gy1)
        c2 = cw * cw + ch * ch + _CEPS
        rho2 = ((gx1 + gx2 - px1 - px2) ** 2
                + (gy1 + gy2 - py1 - py2) ** 2) * 0.25
        dd = at2 - at1
        v = (4.0 / math.pi ** 2) * dd * dd
        alpha = v / (v - iou + (1.0 + _CEPS))
        ciou = iou - (rho2 / c2 + v * alpha)
        ov = jnp.clip(ciou, -1.0, 1.0) * mask

        ovr = jnp.maximum(ov, 0.0)
        ovr2 = ovr * ovr
        align = jnp.sqrt(bsc * mask) * (ovr2 * ovr2 * ovr2)

        al_s[t] = align
        ov_s[t] = ov
        aux_s[t, 0:1, :] = px1
        aux_s[t, 1:2, :] = py1
        aux_s[t, 2:3, :] = px2
        aux_s[t, 3:4, :] = py2
        aux_s[t, 4:5, :] = at1
        aux_s[t, 5:6, :] = logz[0]
        aux_s[t, 6:7, :] = logz[1]
        aux_s[t, 7:8, :] = logz[2]
        aux_s[t, 8:9, :] = logz[3]

        out_ref[...] += jnp.where(lane == 1, sp_sum, 0.0)
        return 0

    jax.lax.fori_loop(0, _NT, pass1, 0)

    work = al_s[...]
    for _ in range(_TOPK - 1):
        cur = jnp.max(work, axis=(0, 2))[None, :, None]
        work = jnp.where(work >= cur, -1.0, work)
    thr = jnp.max(work, axis=(0, 2))[:, None]

    def pass2(t, _):
        align = al_s[t]
        ov = ov_s[t]

        pos0 = jnp.logical_and(align >= thr, align > _EPS).astype(f32)
        fg0 = jnp.sum(pos0, axis=0, keepdims=True)
        multi = fg0 > 1.0

        mxo = jnp.max(ov, axis=0, keepdims=True)
        fidx = jnp.min(jnp.where(ov >= mxo, i32s, float(_NMAX)), axis=0,
                       keepdims=True)
        ismax = (i32s == fidx).astype(f32)
        mask_pos = jnp.where(multi, ismax, pos0)
        mp_s[t] = mask_pos

        am_col = jnp.max(align * mask_pos, axis=1, keepdims=True)
        ov_col = jnp.max(ov * mask_pos, axis=1, keepdims=True)
        cm_s[:, 0:1] = jnp.maximum(cm_s[:, 0:1], am_col)
        cm_s[:, 1:2] = jnp.maximum(cm_s[:, 1:2], ov_col)
        return 0

    jax.lax.fori_loop(0, _NT, pass2, 0)

    pos_align = cm_s[:, 0:1]
    pos_ov = cm_s[:, 1:2]
    ratio = pos_ov / (pos_align + _EPS)

    def pass3(t, _):
        sl = pl.ds(t * _TA, _TA)
        align = al_s[t]
        mask_pos = mp_s[t]
        pd = pdt_s[t]
        ax = anc_ref[t, 0:1, :]
        ay = anc_ref[t, 1:2, :]
        px1 = aux_s[t, 0:1, :]
        py1 = aux_s[t, 1:2, :]
        px2 = aux_s[t, 2:3, :]
        py2 = aux_s[t, 3:4, :]
        at1 = aux_s[t, 4:5, :]

        mxp = jnp.max(mask_pos, axis=0, keepdims=True)
        sidx = jnp.min(jnp.where(mask_pos >= mxp, i32s, float(_NMAX)),
                       axis=0, keepdims=True)
        sel = (i32s == sidx).astype(f32)
        fg = (mxp > 0.0).astype(f32)

        def _sel(col):
            return jnp.sum(sel * col, axis=0, keepdims=True)

        tx1, ty1, tx2, ty2 = _sel(gx1), _sel(gy1), _sel(gx2), _sel(gy2)
        tat2 = _sel(at2)

        ampos = align * mask_pos
        norm = jnp.max(ampos * ratio, axis=0, keepdims=True)

        fgn = fg * norm
        weight = jnp.clip(fgn, 1e-6, None)
        ts_sum = jnp.sum(fgn)

        g = sel * fgn
        gps = jax.lax.dot_general(g, ps_ref[0, sl, :],
                                  (((1,), (0,)), ((), ())),
                                  preferred_element_type=f32)
        neg_bce = jnp.sum(onehot_lab * gps)

        w1 = px2 - px1
        h1 = py2 - py1 + _CEPS
        tw = tx2 - tx1
        th = ty2 - ty1 + _CEPS
        inter = (jnp.maximum(jnp.minimum(px2, tx2) - jnp.maximum(px1, tx1),
                             0.0)
                 * jnp.maximum(jnp.minimum(py2, ty2) - jnp.maximum(py1, ty1),
                               0.0))
        union = w1 * h1 + tw * th - inter + _CEPS
        iou = inter / union
        cw = jnp.maximum(px2, tx2) - jnp.minimum(px1, tx1)
        ch = jnp.maximum(py2, ty2) - jnp.minimum(py1, ty1)
        c2 = cw * cw + ch * ch + _CEPS
        rho2 = ((tx1 + tx2 - px1 - px2) ** 2
                + (ty1 + ty2 - py1 - py2) ** 2) * 0.25
        dd = tat2 - at1
        v = (4.0 / math.pi ** 2) * dd * dd
        alpha = v / (v - iou + (1.0 + _CEPS))
        ciou = iou - (rho2 / c2 + v * alpha)
        iou_t = jnp.clip(ciou, -1.0, 1.0)
        iou_sum = jnp.sum((1.0 - iou_t) * weight * fg)

        tltrb = [jnp.clip(tv, 0.0, _REG_MAX - 1.01)
                 for tv in (ax - tx1, ay - ty1, tx2 - ax, ty2 - ay)]
        acc = jnp.zeros((1, _TA), f32)
        for c in range(4):
            x = pd[c * _REG_MAX:(c + 1) * _REG_MAX, :]
            hat = jnp.maximum(1.0 - jnp.abs(tltrb[c] - i16s), 0.0)
            acc = acc + (aux_s[t, 5 + c:6 + c, :]
                         - jnp.sum(x * hat, axis=0, keepdims=True))
        dfl = jnp.clip(acc * 0.25, None, 100.0)
        dfl_sum = jnp.sum(dfl * weight * fg)

        out_ref[...] += (jnp.where(lane == 0, ts_sum, 0.0)
                         + jnp.where(lane == 1, -neg_bce, 0.0)
                         + jnp.where(lane == 2, iou_sum, 0.0)
                         + jnp.where(lane == 3, dfl_sum, 0.0))
        return 0

    jax.lax.fori_loop(0, _NT, pass3, 0)


def kernel(pred_distri, pred_scores, anchor_points, gt_labels, gt_bboxes,
           mask_gt):
    f32 = jnp.float32
    anc_t = anchor_points.reshape(_NT, _TA, 2).transpose(0, 2, 1)
    lab_f = gt_labels.astype(f32)

    out = pl.pallas_call(
        _loss_kernel,
        grid=(_B,),
        in_specs=[
            pl.BlockSpec((1, _A, 4 * _REG_MAX), lambda b: (b, 0, 0)),
            pl.BlockSpec((1, _A, _NC), lambda b: (b, 0, 0)),
            pl.BlockSpec((_NT, 2, _TA), lambda b: (0, 0, 0)),
            pl.BlockSpec((1, _NMAX, 4), lambda b: (b, 0, 0)),
            pl.BlockSpec((1, _NMAX, 1), lambda b: (b, 0, 0)),
            pl.BlockSpec((1, _NMAX, 1), lambda b: (b, 0, 0)),
        ],
        out_specs=pl.BlockSpec((1, 128), lambda b: (0, 0)),
        out_shape=jax.ShapeDtypeStruct((1, 128), f32),
        scratch_shapes=[
            pltpu.VMEM((_NT, _NMAX, _TA), f32),
            pltpu.VMEM((_NT, _NMAX, _TA), f32),
            pltpu.VMEM((_NT, _NMAX, _TA), f32),
            pltpu.VMEM((_NT, 16, _TA), f32),
            pltpu.VMEM((_NT, 4 * _REG_MAX, _TA), f32),
            pltpu.VMEM((_NMAX, 128), f32),
        ],
    )(pred_distri, pred_scores, anc_t, gt_bboxes, lab_f, mask_gt)

    s = out[0]
    tss = jnp.maximum(s[0], 1.0)
    loss_iou = jnp.clip(s[2] / tss, None, 100.0)
    loss_cls = s[1] / tss
    loss_dfl = jnp.clip(s[3] / tss, None, 100.0)
    return jnp.stack([loss_iou, loss_cls, loss_dfl])


# speedup vs baseline: 24.5372x; 1.0184x over previous
import math

import jax
import jax.numpy as jnp
from jax.experimental import pallas as pl
from jax.experimental.pallas import tpu as pltpu

_B, _A, _NC, _REG_MAX, _NMAX, _TOPK = 16, 8400, 80, 16, 32, 10
_ALPHA, _BETA, _EPS = 0.5, 6.0, 1e-9
_CEPS = 1e-7

_NT = 6
_TA = _A // _NT

_ATAN_COEFFS = (
    9.9999999773e-01, -3.3333285376e-01, 1.9998315719e-01, -1.4262475385e-01,
    1.0944970499e-01, -8.3862066348e-02, 5.7709186551e-02, -3.0965612942e-02,
    1.0815613194e-02, -1.7742115459e-03,
)


def _atan(x):
    a = jnp.abs(x)
    inv = a > 1.0
    r = jnp.where(inv, 1.0 / jnp.maximum(a, 1e-30), a)
    z = r * r
    p = jnp.full_like(z, _ATAN_COEFFS[-1])
    for c in _ATAN_COEFFS[-2::-1]:
        p = p * z + c
    p = p * r
    res = jnp.where(inv, (math.pi / 2) - p, p)
    return jnp.where(x < 0, -res, res)


def _loss_kernel(pd_ref, ps_ref, anc_ref, gtb_ref, lab_ref, mgt_ref, out_ref,
                 al_s, ov_s, mp_s, aux_s, pdt_s, cm_s):
    b = pl.program_id(0)
    f32 = jnp.float32

    gx1 = gtb_ref[0, :, 0:1]
    gy1 = gtb_ref[0, :, 1:2]
    gx2 = gtb_ref[0, :, 2:3]
    gy2 = gtb_ref[0, :, 3:4]
    lab = lab_ref[0]
    mgt = mgt_ref[0]

    w2 = gx2 - gx1
    h2 = gy2 - gy1 + _CEPS
    at2 = _atan(w2 / (h2 + _CEPS))
    w2h2 = w2 * h2
    gsx = gx1 + gx2
    gsy = gy1 + gy2

    i16s = jax.lax.broadcasted_iota(jnp.int32, (_REG_MAX, _TA), 0).astype(f32)
    i32s = jax.lax.broadcasted_iota(jnp.int32, (_NMAX, _TA), 0).astype(f32)
    iota_nc = jax.lax.broadcasted_iota(jnp.int32, (_NMAX, _NC), 1).astype(f32)
    onehot_lab = (iota_nc == lab).astype(f32)
    lane = jax.lax.broadcasted_iota(jnp.int32, (1, 128), 1)

    @pl.when(b == 0)
    def _():
        out_ref[...] = jnp.zeros((1, 128), f32)

    cm_s[...] = jnp.zeros((_NMAX, 128), f32)

    def pass1(t, _):
        sl = pl.ds(t * _TA, _TA)
        pd = jnp.transpose(pd_ref[0, sl, :])
        ps = jnp.transpose(ps_ref[0, sl, :])
        pdt_s[t] = pd
        ax = anc_ref[t, 0:1, :]
        ay = anc_ref[t, 1:2, :]

        ltrb, logz = [], []
        for c in range(4):
            x = pd[c * _REG_MAX:(c + 1) * _REG_MAX, :]
            m = jnp.max(x, axis=0, keepdims=True)
            e = jnp.exp(x - m)
            s = jnp.sum(e, axis=0, keepdims=True)
            ltrb.append(jnp.sum(e * i16s, axis=0, keepdims=True) / s)
            logz.append(m + jnp.log(s))

        px1 = ax - ltrb[0]
        py1 = ay - ltrb[1]
        px2 = ax + ltrb[2]
        py2 = ay + ltrb[3]

        sig = 1.0 / (1.0 + jnp.exp(-ps))
        bsc = jnp.dot(onehot_lab, sig, preferred_element_type=f32)

        sp_sum = jnp.sum(jnp.maximum(ps, 0.0)
                         + jnp.log1p(jnp.exp(-jnp.abs(ps))))

        dmin = jnp.minimum(jnp.minimum(ax - gx1, ay - gy1),
                           jnp.minimum(gx2 - ax, gy2 - ay))
        mask = (dmin > _EPS).astype(f32) * mgt

        w1 = px2 - px1
        h1 = py2 - py1 + _CEPS
        at1 = _atan(w1 / (h1 + _CEPS))
        inter = (jnp.maximum(jnp.minimum(px2, gx2) - jnp.maximum(px1, gx1),
                             0.0)
                 * jnp.maximum(jnp.minimum(py2, gy2) - jnp.maximum(py1, gy1),
                               0.0))
        w1h1 = w1 * h1
        union = w1h1 + w2h2 - inter + _CEPS
        iou = inter / union
        cw = jnp.maximum(px2, gx2) - jnp.minimum(px1, gx1)
        ch = jnp.maximum(py2, gy2) - jnp.minimum(py1, gy1)
        c2 = cw * cw + ch * ch + _CEPS
        psx = px1 + px2
        psy = py1 + py2
        rho2 = ((gsx - psx) ** 2 + (gsy - psy) ** 2) * 0.25
        dd = at2 - at1
        v = (4.0 / math.pi ** 2) * dd * dd
        alpha = v / (v - iou + (1.0 + _CEPS))
        ciou = iou - (rho2 / c2 + v * alpha)
        ov = jnp.clip(ciou, -1.0, 1.0) * mask

        ovr = jnp.maximum(ov, 0.0)
        ovr2 = ovr * ovr
        align = jnp.sqrt(bsc * mask) * (ovr2 * ovr2 * ovr2)

        al_s[t] = align
        ov_s[t] = ov
        aux_s[t, 0:1, :] = px1
        aux_s[t, 1:2, :] = py1
        aux_s[t, 2:3, :] = px2
        aux_s[t, 3:4, :] = py2
        aux_s[t, 4:5, :] = at1
        aux_s[t, 5:6, :] = logz[0]
        aux_s[t, 6:7, :] = logz[1]
        aux_s[t, 7:8, :] = logz[2]
        aux_s[t, 8:9, :] = logz[3]

        out_ref[...] += jnp.where(lane == 1, sp_sum, 0.0)
        return 0

    jax.lax.fori_loop(0, _NT, pass1, 0)

    work = al_s[...]
    cur = jnp.max(work, axis=(0, 2))[None, :, None]
    for _ in range(_TOPK - 1):
        cur = jnp.max(jnp.where(work < cur, work, -1.0),
                      axis=(0, 2))[None, :, None]
    thr = cur[0].reshape(_NMAX, 1)

    def pass2(t, _):
        align = al_s[t]
        ov = ov_s[t]

        pos0 = jnp.logical_and(align >= thr, align > _EPS).astype(f32)
        fg0 = jnp.sum(pos0, axis=0, keepdims=True)
        multi = fg0 > 1.0

        mxo = jnp.max(ov, axis=0, keepdims=True)
        fidx = jnp.min(jnp.where(ov >= mxo, i32s, float(_NMAX)), axis=0,
                       keepdims=True)
        ismax = (i32s == fidx).astype(f32)
        mask_pos = jnp.where(multi, ismax, pos0)
        mp_s[t] = mask_pos

        am_col = jnp.max(align * mask_pos, axis=1, keepdims=True)
        ov_col = jnp.max(ov * mask_pos, axis=1, keepdims=True)
        cm_s[:, 0:1] = jnp.maximum(cm_s[:, 0:1], am_col)
        cm_s[:, 1:2] = jnp.maximum(cm_s[:, 1:2], ov_col)
        return 0

    jax.lax.fori_loop(0, _NT, pass2, 0)

    pos_align = cm_s[:, 0:1]
    pos_ov = cm_s[:, 1:2]
    ratio = pos_ov / (pos_align + _EPS)

    def pass3(t, _):
        sl = pl.ds(t * _TA, _TA)
        align = al_s[t]
        mask_pos = mp_s[t]
        pd = pdt_s[t]
        ax = anc_ref[t, 0:1, :]
        ay = anc_ref[t, 1:2, :]
        px1 = aux_s[t, 0:1, :]
        py1 = aux_s[t, 1:2, :]
        px2 = aux_s[t, 2:3, :]
        py2 = aux_s[t, 3:4, :]
        at1 = aux_s[t, 4:5, :]

        mxp = jnp.max(mask_pos, axis=0, keepdims=True)
        sidx = jnp.min(jnp.where(mask_pos >= mxp, i32s, float(_NMAX)),
                       axis=0, keepdims=True)
        sel = (i32s == sidx).astype(f32)
        fg = (mxp > 0.0).astype(f32)

        def _sel(col):
            return jnp.sum(sel * col, axis=0, keepdims=True)

        tx1, ty1, tx2, ty2 = _sel(gx1), _sel(gy1), _sel(gx2), _sel(gy2)
        tat2 = _sel(at2)

        ampos = align * mask_pos
        norm = jnp.max(ampos * ratio, axis=0, keepdims=True)

        fgn = fg * norm
        weight = jnp.clip(fgn, 1e-6, None)
        ts_sum = jnp.sum(fgn)

        g = sel * fgn
        gps = jax.lax.dot_general(g, ps_ref[0, sl, :],
                                  (((1,), (0,)), ((), ())),
                                  preferred_element_type=f32)
        neg_bce = jnp.sum(onehot_lab * gps)

        w1 = px2 - px1
        h1 = py2 - py1 + _CEPS
        tw = tx2 - tx1
        th = ty2 - ty1 + _CEPS
        inter = (jnp.maximum(jnp.minimum(px2, tx2) - jnp.maximum(px1, tx1),
                             0.0)
                 * jnp.maximum(jnp.minimum(py2, ty2) - jnp.maximum(py1, ty1),
                               0.0))
        union = w1 * h1 + tw * th - inter + _CEPS
        iou = inter / union
        cw = jnp.maximum(px2, tx2) - jnp.minimum(px1, tx1)
        ch = jnp.maximum(py2, ty2) - jnp.minimum(py1, ty1)
        c2 = cw * cw + ch * ch + _CEPS
        rho2 = ((tx1 + tx2 - px1 - px2) ** 2
                + (ty1 + ty2 - py1 - py2) ** 2) * 0.25
        dd = tat2 - at1
        v = (4.0 / math.pi ** 2) * dd * dd
        alpha = v / (v - iou + (1.0 + _CEPS))
        ciou = iou - (rho2 / c2 + v * alpha)
        iou_t = jnp.clip(ciou, -1.0, 1.0)
        iou_sum = jnp.sum((1.0 - iou_t) * weight * fg)

        tltrb = [jnp.clip(tv, 0.0, _REG_MAX - 1.01)
                 for tv in (ax - tx1, ay - ty1, tx2 - ax, ty2 - ay)]
        acc = jnp.zeros((1, _TA), f32)
        for c in range(4):
            x = pd[c * _REG_MAX:(c + 1) * _REG_MAX, :]
            hat = jnp.maximum(1.0 - jnp.abs(tltrb[c] - i16s), 0.0)
            acc = acc + (aux_s[t, 5 + c:6 + c, :]
                         - jnp.sum(x * hat, axis=0, keepdims=True))
        dfl = jnp.clip(acc * 0.25, None, 100.0)
        dfl_sum = jnp.sum(dfl * weight * fg)

        out_ref[...] += (jnp.where(lane == 0, ts_sum, 0.0)
                         + jnp.where(lane == 1, -neg_bce, 0.0)
                         + jnp.where(lane == 2, iou_sum, 0.0)
                         + jnp.where(lane == 3, dfl_sum, 0.0))
        return 0

    jax.lax.fori_loop(0, _NT, pass3, 0)


def kernel(pred_distri, pred_scores, anchor_points, gt_labels, gt_bboxes,
           mask_gt):
    f32 = jnp.float32
    anc_t = anchor_points.reshape(_NT, _TA, 2).transpose(0, 2, 1)
    lab_f = gt_labels.astype(f32)

    out = pl.pallas_call(
        _loss_kernel,
        grid=(_B,),
        in_specs=[
            pl.BlockSpec((1, _A, 4 * _REG_MAX), lambda b: (b, 0, 0)),
            pl.BlockSpec((1, _A, _NC), lambda b: (b, 0, 0)),
            pl.BlockSpec((_NT, 2, _TA), lambda b: (0, 0, 0)),
            pl.BlockSpec((1, _NMAX, 4), lambda b: (b, 0, 0)),
            pl.BlockSpec((1, _NMAX, 1), lambda b: (b, 0, 0)),
            pl.BlockSpec((1, _NMAX, 1), lambda b: (b, 0, 0)),
        ],
        out_specs=pl.BlockSpec((1, 128), lambda b: (0, 0)),
        out_shape=jax.ShapeDtypeStruct((1, 128), f32),
        scratch_shapes=[
            pltpu.VMEM((_NT, _NMAX, _TA), f32),
            pltpu.VMEM((_NT, _NMAX, _TA), f32),
            pltpu.VMEM((_NT, _NMAX, _TA), f32),
            pltpu.VMEM((_NT, 16, _TA), f32),
            pltpu.VMEM((_NT, 4 * _REG_MAX, _TA), f32),
            pltpu.VMEM((_NMAX, 128), f32),
        ],
    )(pred_distri, pred_scores, anc_t, gt_bboxes, lab_f, mask_gt)

    s = out[0]
    tss = jnp.maximum(s[0], 1.0)
    loss_iou = jnp.clip(s[2] / tss, None, 100.0)
    loss_cls = s[1] / tss
    loss_dfl = jnp.clip(s[3] / tss, None, 100.0)
    return jnp.stack([loss_iou, loss_cls, loss_dfl])


# speedup vs baseline: 25.4013x; 1.0352x over previous
import math

import jax
import jax.numpy as jnp
from jax.experimental import pallas as pl
from jax.experimental.pallas import tpu as pltpu

_B, _A, _NC, _REG_MAX, _NMAX, _TOPK = 16, 8400, 80, 16, 32, 10
_ALPHA, _BETA, _EPS = 0.5, 6.0, 1e-9
_CEPS = 1e-7

_NT = 6
_TA = _A // _NT

_ATAN_COEFFS = (
    9.9999999773e-01, -3.3333285376e-01, 1.9998315719e-01, -1.4262475385e-01,
    1.0944970499e-01, -8.3862066348e-02, 5.7709186551e-02, -3.0965612942e-02,
    1.0815613194e-02, -1.7742115459e-03,
)


def _atan(x):
    a = jnp.abs(x)
    inv = a > 1.0
    r = jnp.where(inv, 1.0 / jnp.maximum(a, 1e-30), a)
    z = r * r
    p = jnp.full_like(z, _ATAN_COEFFS[-1])
    for c in _ATAN_COEFFS[-2::-1]:
        p = p * z + c
    p = p * r
    res = jnp.where(inv, (math.pi / 2) - p, p)
    return jnp.where(x < 0, -res, res)


def _mm(a, b):
    return jnp.dot(a, b, preferred_element_type=jnp.float32)


def _loss_kernel(pd_ref, ps_ref, anc_ref, gtb_ref, lab_ref, mgt_ref, out_ref,
                 al_s, ov_s, mp_s, aux_s, pdt_s, cm_s, row_s):
    b = pl.program_id(0)
    f32 = jnp.float32

    gx1 = gtb_ref[0, :, 0:1]
    gy1 = gtb_ref[0, :, 1:2]
    gx2 = gtb_ref[0, :, 2:3]
    gy2 = gtb_ref[0, :, 3:4]
    lab = lab_ref[0]
    mgt = mgt_ref[0]

    w2 = gx2 - gx1
    h2 = gy2 - gy1 + _CEPS
    at2 = _atan(w2 / (h2 + _CEPS))
    w2h2 = w2 * h2
    gsx = gx1 + gx2
    gsy = gy1 + gy2

    i16s = jax.lax.broadcasted_iota(jnp.int32, (_REG_MAX, _TA), 0).astype(f32)
    i32s = jax.lax.broadcasted_iota(jnp.int32, (_NMAX, _TA), 0).astype(f32)
    iota_nc = jax.lax.broadcasted_iota(jnp.int32, (_NMAX, _NC), 1).astype(f32)
    onehot_lab = (iota_nc == lab).astype(f32)
    lane = jax.lax.broadcasted_iota(jnp.int32, (1, 128), 1)

    r8 = jax.lax.broadcasted_iota(jnp.int32, (8, 4 * _REG_MAX), 0)
    k8 = jax.lax.broadcasted_iota(jnp.int32, (8, 4 * _REG_MAX), 1)
    grp = k8 // _REG_MAX
    m8 = jnp.where(r8 < 4, (grp == r8).astype(f32),
                   (grp == r8 - 4).astype(f32)
                   * (k8 - grp * _REG_MAX).astype(f32))

    gt4 = jnp.transpose(gtb_ref[0])
    gmat = jnp.concatenate(
        [gt4, jnp.transpose(at2), jnp.zeros((3, _NMAX), f32)], axis=0)

    ones32 = jnp.ones((1, _NMAX), f32)
    ones80 = jnp.ones((1, _NC), f32)

    @pl.when(b == 0)
    def _():
        out_ref[...] = jnp.zeros((1, 128), f32)

    cm_s[...] = jnp.zeros((_NMAX, 128), f32)
    row_s[...] = jnp.zeros((8, _TA), f32)

    def pass1(t, _):
        sl = pl.ds(t * _TA, _TA)
        pd = jnp.transpose(pd_ref[0, sl, :])
        ps = jnp.transpose(ps_ref[0, sl, :])
        pdt_s[t] = pd
        ax = anc_ref[t, 0:1, :]
        ay = anc_ref[t, 1:2, :]

        ms, es = [], []
        for c in range(4):
            x = pd[c * _REG_MAX:(c + 1) * _REG_MAX, :]
            m = jnp.max(x, axis=0, keepdims=True)
            ms.append(m)
            es.append(jnp.exp(x - m))
        e64 = jnp.concatenate(es, axis=0)
        sw = _mm(m8, e64)
        ltrb = [sw[4 + c:5 + c, :] / sw[c:c + 1, :] for c in range(4)]
        logz = [ms[c] + jnp.log(sw[c:c + 1, :]) for c in range(4)]

        px1 = ax - ltrb[0]
        py1 = ay - ltrb[1]
        px2 = ax + ltrb[2]
        py2 = ay + ltrb[3]

        sig = 1.0 / (1.0 + jnp.exp(-ps))
        bsc = _mm(onehot_lab, sig)

        sp = jnp.maximum(ps, 0.0) + jnp.log1p(jnp.exp(-jnp.abs(ps)))
        row_s[3:4, :] += _mm(ones80, sp)

        dmin = jnp.minimum(jnp.minimum(ax - gx1, ay - gy1),
                           jnp.minimum(gx2 - ax, gy2 - ay))
        mask = (dmin > _EPS).astype(f32) * mgt

        w1 = px2 - px1
        h1 = py2 - py1 + _CEPS
        at1 = _atan(w1 / (h1 + _CEPS))
        inter = (jnp.maximum(jnp.minimum(px2, gx2) - jnp.maximum(px1, gx1),
                             0.0)
                 * jnp.maximum(jnp.minimum(py2, gy2) - jnp.maximum(py1, gy1),
                               0.0))
        w1h1 = w1 * h1
        union = w1h1 + w2h2 - inter + _CEPS
        iou = inter / union
        cw = jnp.maximum(px2, gx2) - jnp.minimum(px1, gx1)
        ch = jnp.maximum(py2, gy2) - jnp.minimum(py1, gy1)
        c2 = cw * cw + ch * ch + _CEPS
        psx = px1 + px2
        psy = py1 + py2
        rho2 = ((gsx - psx) ** 2 + (gsy - psy) ** 2) * 0.25
        dd = at2 - at1
        v = (4.0 / math.pi ** 2) * dd * dd
        alpha = v / (v - iou + (1.0 + _CEPS))
        ciou = iou - (rho2 / c2 + v * alpha)
        ov = jnp.clip(ciou, -1.0, 1.0) * mask

        ovr = jnp.maximum(ov, 0.0)
        ovr2 = ovr * ovr
        align = jnp.sqrt(bsc * mask) * (ovr2 * ovr2 * ovr2)

        al_s[t] = align
        ov_s[t] = ov
        aux_s[t, 0:1, :] = px1
        aux_s[t, 1:2, :] = py1
        aux_s[t, 2:3, :] = px2
        aux_s[t, 3:4, :] = py2
        aux_s[t, 4:5, :] = at1
        aux_s[t, 5:6, :] = logz[0]
        aux_s[t, 6:7, :] = logz[1]
        aux_s[t, 7:8, :] = logz[2]
        aux_s[t, 8:9, :] = logz[3]
        return 0

    jax.lax.fori_loop(0, _NT, pass1, 0)

    work = al_s[...]
    cur = jnp.max(work, axis=(0, 2))[None, :, None]
    for _ in range(_TOPK - 1):
        cur = jnp.max(jnp.where(work < cur, work, -1.0),
                      axis=(0, 2))[None, :, None]
    thr = cur[0].reshape(_NMAX, 1)

    def pass2(t, _):
        align = al_s[t]
        ov = ov_s[t]

        pos0 = jnp.logical_and(align >= thr, align > _EPS).astype(f32)
        fg0 = _mm(ones32, pos0)
        multi = fg0 > 1.0

        mxo = jnp.max(ov, axis=0, keepdims=True)
        fidx = jnp.min(jnp.where(ov >= mxo, i32s, float(_NMAX)), axis=0,
                       keepdims=True)
        ismax = (i32s == fidx).astype(f32)
        mask_pos = jnp.where(multi, ismax, pos0)
        mp_s[t] = mask_pos

        am_col = jnp.max(align * mask_pos, axis=1, keepdims=True)
        ov_col = jnp.max(ov * mask_pos, axis=1, keepdims=True)
        cm_s[:, 0:1] = jnp.maximum(cm_s[:, 0:1], am_col)
        cm_s[:, 1:2] = jnp.maximum(cm_s[:, 1:2], ov_col)
        return 0

    jax.lax.fori_loop(0, _NT, pass2, 0)

    pos_align = cm_s[:, 0:1]
    pos_ov = cm_s[:, 1:2]
    ratio = pos_ov / (pos_align + _EPS)

    def pass3(t, _):
        sl = pl.ds(t * _TA, _TA)
        align = al_s[t]
        mask_pos = mp_s[t]
        pd = pdt_s[t]
        ax = anc_ref[t, 0:1, :]
        ay = anc_ref[t, 1:2, :]
        px1 = aux_s[t, 0:1, :]
        py1 = aux_s[t, 1:2, :]
        px2 = aux_s[t, 2:3, :]
        py2 = aux_s[t, 3:4, :]
        at1 = aux_s[t, 4:5, :]

        mxp = jnp.max(mask_pos, axis=0, keepdims=True)
        sidx = jnp.min(jnp.where(mask_pos >= mxp, i32s, float(_NMAX)),
                       axis=0, keepdims=True)
        sel = (i32s == sidx).astype(f32)
        fg = (mxp > 0.0).astype(f32)

        tsel = _mm(gmat, sel)
        tx1 = tsel[0:1, :]
        ty1 = tsel[1:2, :]
        tx2 = tsel[2:3, :]
        ty2 = tsel[3:4, :]
        tat2 = tsel[4:5, :]

        ampos = align * mask_pos
        norm = jnp.max(ampos * ratio, axis=0, keepdims=True)

        fgn = fg * norm
        weight = jnp.clip(fgn, 1e-6, None)
        wf = weight * fg

        g = sel * fgn
        gps = jax.lax.dot_general(g, ps_ref[0, sl, :],
                                  (((1,), (0,)), ((), ())),
                                  preferred_element_type=f32)
        neg_bce = jnp.sum(onehot_lab * gps)

        w1 = px2 - px1
        h1 = py2 - py1 + _CEPS
        tw = tx2 - tx1
        th = ty2 - ty1 + _CEPS
        inter = (jnp.maximum(jnp.minimum(px2, tx2) - jnp.maximum(px1, tx1),
                             0.0)
                 * jnp.maximum(jnp.minimum(py2, ty2) - jnp.maximum(py1, ty1),
                               0.0))
        union = w1 * h1 + tw * th - inter + _CEPS
        iou = inter / union
        cw = jnp.maximum(px2, tx2) - jnp.minimum(px1, tx1)
        ch = jnp.maximum(py2, ty2) - jnp.minimum(py1, ty1)
        c2 = cw * cw + ch * ch + _CEPS
        rho2 = ((tx1 + tx2 - px1 - px2) ** 2
                + (ty1 + ty2 - py1 - py2) ** 2) * 0.25
        dd = tat2 - at1
        v = (4.0 / math.pi ** 2) * dd * dd
        alpha = v / (v - iou + (1.0 + _CEPS))
        ciou = iou - (rho2 / c2 + v * alpha)
        iou_t = jnp.clip(ciou, -1.0, 1.0)

        tltrb = [jnp.clip(tv, 0.0, _REG_MAX - 1.01)
                 for tv in (ax - tx1, ay - ty1, tx2 - ax, ty2 - ay)]
        acc = jnp.zeros((1, _TA), f32)
        for c in range(4):
            x = pd[c * _REG_MAX:(c + 1) * _REG_MAX, :]
            hat = jnp.maximum(1.0 - jnp.abs(tltrb[c] - i16s), 0.0)
            acc = acc + (aux_s[t, 5 + c:6 + c, :]
                         - jnp.sum(x * hat, axis=0, keepdims=True))
        dfl = jnp.clip(acc * 0.25, None, 100.0)

        row_s[0:1, :] += fgn
        row_s[1:2, :] += (1.0 - iou_t) * wf
        row_s[2:3, :] += dfl * wf
        out_ref[...] += jnp.where(lane == 1, -neg_bce, 0.0)
        return 0

    jax.lax.fori_loop(0, _NT, pass3, 0)

    sums = jnp.sum(row_s[...], axis=1, keepdims=True)
    out_ref[...] += (jnp.where(lane == 0, sums[0, 0], 0.0)
                     + jnp.where(lane == 1, sums[3, 0], 0.0)
                     + jnp.where(lane == 2, sums[1, 0], 0.0)
                     + jnp.where(lane == 3, sums[2, 0], 0.0))


def kernel(pred_distri, pred_scores, anchor_points, gt_labels, gt_bboxes,
           mask_gt):
    f32 = jnp.float32
    anc_t = anchor_points.reshape(_NT, _TA, 2).transpose(0, 2, 1)
    lab_f = gt_labels.astype(f32)

    out = pl.pallas_call(
        _loss_kernel,
        grid=(_B,),
        in_specs=[
            pl.BlockSpec((1, _A, 4 * _REG_MAX), lambda b: (b, 0, 0)),
            pl.BlockSpec((1, _A, _NC), lambda b: (b, 0, 0)),
            pl.BlockSpec((_NT, 2, _TA), lambda b: (0, 0, 0)),
            pl.BlockSpec((1, _NMAX, 4), lambda b: (b, 0, 0)),
            pl.BlockSpec((1, _NMAX, 1), lambda b: (b, 0, 0)),
            pl.BlockSpec((1, _NMAX, 1), lambda b: (b, 0, 0)),
        ],
        out_specs=pl.BlockSpec((1, 128), lambda b: (0, 0)),
        out_shape=jax.ShapeDtypeStruct((1, 128), f32),
        scratch_shapes=[
            pltpu.VMEM((_NT, _NMAX, _TA), f32),
            pltpu.VMEM((_NT, _NMAX, _TA), f32),
            pltpu.VMEM((_NT, _NMAX, _TA), f32),
            pltpu.VMEM((_NT, 16, _TA), f32),
            pltpu.VMEM((_NT, 4 * _REG_MAX, _TA), f32),
            pltpu.VMEM((_NMAX, 128), f32),
            pltpu.VMEM((8, _TA), f32),
        ],
    )(pred_distri, pred_scores, anc_t, gt_bboxes, lab_f, mask_gt)

    s = out[0]
    tss = jnp.maximum(s[0], 1.0)
    loss_iou = jnp.clip(s[2] / tss, None, 100.0)
    loss_cls = s[1] / tss
    loss_dfl = jnp.clip(s[3] / tss, None, 100.0)
    return jnp.stack([loss_iou, loss_cls, loss_dfl])


# speedup vs baseline: 36.2580x; 1.4274x over previous
import math

import jax
import jax.numpy as jnp
from jax.experimental import pallas as pl
from jax.experimental.pallas import tpu as pltpu

_B, _A, _NC, _REG_MAX, _NMAX, _TOPK = 16, 8400, 80, 16, 32, 10
_ALPHA, _BETA, _EPS = 0.5, 6.0, 1e-9
_CEPS = 1e-7

_NT = 6
_TA = _A // _NT

_ATAN_COEFFS = (
    9.9999999773e-01, -3.3333285376e-01, 1.9998315719e-01, -1.4262475385e-01,
    1.0944970499e-01, -8.3862066348e-02, 5.7709186551e-02, -3.0965612942e-02,
    1.0815613194e-02, -1.7742115459e-03,
)


def _atan(x):
    a = jnp.abs(x)
    inv = a > 1.0
    r = jnp.where(inv, 1.0 / jnp.maximum(a, 1e-30), a)
    z = r * r
    p = jnp.full_like(z, _ATAN_COEFFS[-1])
    for c in _ATAN_COEFFS[-2::-1]:
        p = p * z + c
    p = p * r
    res = jnp.where(inv, (math.pi / 2) - p, p)
    return jnp.where(x < 0, -res, res)


def _mm(a, b):
    return jnp.dot(a, b, preferred_element_type=jnp.float32)


def _loss_kernel(pd_ref, ps_ref, anc_ref, gtb_ref, lab_ref, mgt_ref, out_ref,
                 al_s, ov_s, mp_s, cm_s, row_s):
    b = pl.program_id(0)
    f32 = jnp.float32

    gx1 = gtb_ref[0, :, 0:1]
    gy1 = gtb_ref[0, :, 1:2]
    gx2 = gtb_ref[0, :, 2:3]
    gy2 = gtb_ref[0, :, 3:4]
    lab = lab_ref[0]
    mgt = mgt_ref[0]

    w2 = gx2 - gx1
    h2 = gy2 - gy1 + _CEPS
    at2 = _atan(w2 / (h2 + _CEPS))
    w2h2 = w2 * h2
    gsx = gx1 + gx2
    gsy = gy1 + gy2

    i16s = jax.lax.broadcasted_iota(jnp.int32, (_REG_MAX, _TA), 0).astype(f32)
    i32s = jax.lax.broadcasted_iota(jnp.int32, (_NMAX, _TA), 0).astype(f32)
    iota_nc = jax.lax.broadcasted_iota(jnp.int32, (_NMAX, _NC), 1).astype(f32)
    onehot_lab = (iota_nc == lab).astype(f32)
    lane = jax.lax.broadcasted_iota(jnp.int32, (1, 128), 1)

    r8 = jax.lax.broadcasted_iota(jnp.int32, (8, 4 * _REG_MAX), 0)
    k8 = jax.lax.broadcasted_iota(jnp.int32, (8, 4 * _REG_MAX), 1)
    grp = k8 // _REG_MAX
    m8 = jnp.where(r8 < 4, (grp == r8).astype(f32),
                   (grp == r8 - 4).astype(f32)
                   * (k8 - grp * _REG_MAX).astype(f32))

    gt4 = jnp.transpose(gtb_ref[0])
    gmat = jnp.concatenate(
        [gt4, jnp.transpose(at2), jnp.zeros((3, _NMAX), f32)], axis=0)

    ones32 = jnp.ones((1, _NMAX), f32)
    ones80 = jnp.ones((1, _NC), f32)

    @pl.when(b == 0)
    def _():
        out_ref[...] = jnp.zeros((1, 128), f32)

    cm_s[...] = jnp.zeros((_NMAX, 128), f32)
    row_s[...] = jnp.zeros((8, _TA), f32)

    def pass1(t, _):
        sl = pl.ds(t * _TA, _TA)
        pd = jnp.transpose(pd_ref[0, sl, :])
        ps = jnp.transpose(ps_ref[0, sl, :])
        ax = anc_ref[t, 0:1, :]
        ay = anc_ref[t, 1:2, :]

        ms, es = [], []
        for c in range(4):
            x = pd[c * _REG_MAX:(c + 1) * _REG_MAX, :]
            m = jnp.max(x, axis=0, keepdims=True)
            ms.append(m)
            es.append(jnp.exp(x - m))
        e64 = jnp.concatenate(es, axis=0)
        sw = _mm(m8, e64)
        ltrb = [sw[4 + c:5 + c, :] / sw[c:c + 1, :] for c in range(4)]
        logz = [ms[c] + jnp.log(sw[c:c + 1, :]) for c in range(4)]

        px1 = ax - ltrb[0]
        py1 = ay - ltrb[1]
        px2 = ax + ltrb[2]
        py2 = ay + ltrb[3]

        sig = 1.0 / (1.0 + jnp.exp(-ps))
        bsc = _mm(onehot_lab, sig)

        sp = (jnp.maximum(ps, 0.0)
              - jnp.log(jnp.where(ps >= 0.0, sig, 1.0 - sig)))
        row_s[3:4, :] += _mm(ones80, sp)

        dmin = jnp.minimum(jnp.minimum(ax - gx1, ay - gy1),
                           jnp.minimum(gx2 - ax, gy2 - ay))
        mask = (dmin > _EPS).astype(f32) * mgt

        w1 = px2 - px1
        h1 = py2 - py1 + _CEPS
        at1 = _atan(w1 / (h1 + _CEPS))
        inter = (jnp.maximum(jnp.minimum(px2, gx2) - jnp.maximum(px1, gx1),
                             0.0)
                 * jnp.maximum(jnp.minimum(py2, gy2) - jnp.maximum(py1, gy1),
                               0.0))
        w1h1 = w1 * h1
        union = w1h1 + w2h2 - inter + _CEPS
        iou = inter / union
        cw = jnp.maximum(px2, gx2) - jnp.minimum(px1, gx1)
        ch = jnp.maximum(py2, gy2) - jnp.minimum(py1, gy1)
        c2 = cw * cw + ch * ch + _CEPS
        psx = px1 + px2
        psy = py1 + py2
        rho2 = ((gsx - psx) ** 2 + (gsy - psy) ** 2) * 0.25
        dd = at2 - at1
        v = (4.0 / math.pi ** 2) * dd * dd
        alpha = v / (v - iou + (1.0 + _CEPS))
        ciou = iou - (rho2 / c2 + v * alpha)
        ov = jnp.clip(ciou, -1.0, 1.0) * mask

        ovr = jnp.maximum(ov, 0.0)
        ovr2 = ovr * ovr
        align = jnp.sqrt(bsc * mask) * (ovr2 * ovr2 * ovr2)

        al_s[t] = align
        ov_s[t] = ov
        cm_s[:, 3:4] = jnp.maximum(cm_s[:, 3:4],
                                   jnp.max(align, axis=1, keepdims=True))
        return 0

    jax.lax.fori_loop(0, _NT, pass1, 0)

    amax = jnp.max(cm_s[:, 3:4])

    @pl.when(amax > _EPS)
    def _assigner_phase():
        work = al_s[...]
        cur = cm_s[:, 3:4][None]
        for _ in range(_TOPK - 1):
            cur = jnp.max(jnp.where(work < cur, work, -1.0),
                          axis=(0, 2))[None, :, None]
        thr = cur[0].reshape(_NMAX, 1)

        def pass2(t, _):
            align = al_s[t]
            ov = ov_s[t]

            pos0 = jnp.logical_and(align >= thr, align > _EPS).astype(f32)
            fg0 = _mm(ones32, pos0)
            multi = fg0 > 1.0

            mxo = jnp.max(ov, axis=0, keepdims=True)
            fidx = jnp.min(jnp.where(ov >= mxo, i32s, float(_NMAX)), axis=0,
                           keepdims=True)
            ismax = (i32s == fidx).astype(f32)
            mask_pos = jnp.where(multi, ismax, pos0)
            mp_s[t] = mask_pos

            am_col = jnp.max(align * mask_pos, axis=1, keepdims=True)
            ov_col = jnp.max(ov * mask_pos, axis=1, keepdims=True)
            cm_s[:, 0:1] = jnp.maximum(cm_s[:, 0:1], am_col)
            cm_s[:, 1:2] = jnp.maximum(cm_s[:, 1:2], ov_col)
            return 0

        jax.lax.fori_loop(0, _NT, pass2, 0)

        pos_align = cm_s[:, 0:1]
        pos_ov = cm_s[:, 1:2]
        ratio = pos_ov / (pos_align + _EPS)

        def pass3(t, _):
            sl = pl.ds(t * _TA, _TA)
            align = al_s[t]
            mask_pos = mp_s[t]
            pd = jnp.transpose(pd_ref[0, sl, :])
            ax = anc_ref[t, 0:1, :]
            ay = anc_ref[t, 1:2, :]

            ms, es = [], []
            for c in range(4):
                x = pd[c * _REG_MAX:(c + 1) * _REG_MAX, :]
                m = jnp.max(x, axis=0, keepdims=True)
                ms.append(m)
                es.append(jnp.exp(x - m))
            e64 = jnp.concatenate(es, axis=0)
            sw = _mm(m8, e64)
            ltrb = [sw[4 + c:5 + c, :] / sw[c:c + 1, :] for c in range(4)]
            logz = [ms[c] + jnp.log(sw[c:c + 1, :]) for c in range(4)]
            px1 = ax - ltrb[0]
            py1 = ay - ltrb[1]
            px2 = ax + ltrb[2]
            py2 = ay + ltrb[3]
            at1 = _atan((px2 - px1) / ((py2 - py1 + _CEPS) + _CEPS))

            mxp = jnp.max(mask_pos, axis=0, keepdims=True)
            sidx = jnp.min(jnp.where(mask_pos >= mxp, i32s, float(_NMAX)),
                           axis=0, keepdims=True)
            sel = (i32s == sidx).astype(f32)
            fg = (mxp > 0.0).astype(f32)

            tsel = _mm(gmat, sel)
            tx1 = tsel[0:1, :]
            ty1 = tsel[1:2, :]
            tx2 = tsel[2:3, :]
            ty2 = tsel[3:4, :]
            tat2 = tsel[4:5, :]

            ampos = align * mask_pos
            norm = jnp.max(ampos * ratio, axis=0, keepdims=True)

            fgn = fg * norm
            weight = jnp.clip(fgn, 1e-6, None)
            wf = weight * fg

            g = sel * fgn
            gps = jax.lax.dot_general(g, ps_ref[0, sl, :],
                                      (((1,), (0,)), ((), ())),
                                      preferred_element_type=f32)
            neg_bce = jnp.sum(onehot_lab * gps)

            w1 = px2 - px1
            h1 = py2 - py1 + _CEPS
            tw = tx2 - tx1
            th = ty2 - ty1 + _CEPS
            inter = (jnp.maximum(jnp.minimum(px2, tx2) - jnp.maximum(px1, tx1),
                                 0.0)
                     * jnp.maximum(jnp.minimum(py2, ty2) - jnp.maximum(py1, ty1),
                                   0.0))
            union = w1 * h1 + tw * th - inter + _CEPS
            iou = inter / union
            cw = jnp.maximum(px2, tx2) - jnp.minimum(px1, tx1)
            ch = jnp.maximum(py2, ty2) - jnp.minimum(py1, ty1)
            c2 = cw * cw + ch * ch + _CEPS
            rho2 = ((tx1 + tx2 - px1 - px2) ** 2
                    + (ty1 + ty2 - py1 - py2) ** 2) * 0.25
            dd = tat2 - at1
            v = (4.0 / math.pi ** 2) * dd * dd
            alpha = v / (v - iou + (1.0 + _CEPS))
            ciou = iou - (rho2 / c2 + v * alpha)
            iou_t = jnp.clip(ciou, -1.0, 1.0)

            tltrb = [jnp.clip(tv, 0.0, _REG_MAX - 1.01)
                     for tv in (ax - tx1, ay - ty1, tx2 - ax, ty2 - ay)]
            acc = jnp.zeros((1, _TA), f32)
            for c in range(4):
                x = pd[c * _REG_MAX:(c + 1) * _REG_MAX, :]
                hat = jnp.maximum(1.0 - jnp.abs(tltrb[c] - i16s), 0.0)
                acc = acc + (logz[c]
                             - jnp.sum(x * hat, axis=0, keepdims=True))
            dfl = jnp.clip(acc * 0.25, None, 100.0)

            row_s[0:1, :] += fgn
            row_s[1:2, :] += (1.0 - iou_t) * wf
            row_s[2:3, :] += dfl * wf
            out_ref[...] += jnp.where(lane == 1, -neg_bce, 0.0)
            return 0

        jax.lax.fori_loop(0, _NT, pass3, 0)


    sums = jnp.sum(row_s[...], axis=1, keepdims=True)
    out_ref[...] += (jnp.where(lane == 0, sums[0, 0], 0.0)
                     + jnp.where(lane == 1, sums[3, 0], 0.0)
                     + jnp.where(lane == 2, sums[1, 0], 0.0)
                     + jnp.where(lane == 3, sums[2, 0], 0.0))


def kernel(pred_distri, pred_scores, anchor_points, gt_labels, gt_bboxes,
           mask_gt):
    f32 = jnp.float32
    anc_t = anchor_points.reshape(_NT, _TA, 2).transpose(0, 2, 1)
    lab_f = gt_labels.astype(f32)

    out = pl.pallas_call(
        _loss_kernel,
        grid=(_B,),
        in_specs=[
            pl.BlockSpec((1, _A, 4 * _REG_MAX), lambda b: (b, 0, 0)),
            pl.BlockSpec((1, _A, _NC), lambda b: (b, 0, 0)),
            pl.BlockSpec((_NT, 2, _TA), lambda b: (0, 0, 0)),
            pl.BlockSpec((1, _NMAX, 4), lambda b: (b, 0, 0)),
            pl.BlockSpec((1, _NMAX, 1), lambda b: (b, 0, 0)),
            pl.BlockSpec((1, _NMAX, 1), lambda b: (b, 0, 0)),
        ],
        out_specs=pl.BlockSpec((1, 128), lambda b: (0, 0)),
        out_shape=jax.ShapeDtypeStruct((1, 128), f32),
        scratch_shapes=[
            pltpu.VMEM((_NT, _NMAX, _TA), f32),
            pltpu.VMEM((_NT, _NMAX, _TA), f32),
            pltpu.VMEM((_NT, _NMAX, _TA), f32),
            pltpu.VMEM((_NMAX, 128), f32),
            pltpu.VMEM((8, _TA), f32),
        ],
    )(pred_distri, pred_scores, anc_t, gt_bboxes, lab_f, mask_gt)

    s = out[0]
    tss = jnp.maximum(s[0], 1.0)
    loss_iou = jnp.clip(s[2] / tss, None, 100.0)
    loss_cls = s[1] / tss
    loss_dfl = jnp.clip(s[3] / tss, None, 100.0)
    return jnp.stack([loss_iou, loss_cls, loss_dfl])
